# Initial kernel scaffold; baseline (speedup 1.0000x reference)
import jax
import jax.numpy as jnp
from jax.experimental import pallas as pl


def kernel(features, edge_index, aspect_indices, edge_types, W1, W1_root, b1, W2, W2_root, b2, Wc, bc):
    raise NotImplementedError("write your pallas kernel here")


# speedup vs baseline: 16.4463x; 16.4463x over previous
import functools

import jax
import jax.numpy as jnp
from jax import lax
from jax.experimental import pallas as pl
from jax.experimental.pallas import tpu as pltpu
from jax.experimental.pallas import tpu_sc as plsc

NC = 2
NS = 16
NW = NC * NS
K = 128
ZCH = 64


def _mesh():
    return plsc.VectorSubcoreMesh(core_axis_name="c", subcore_axis_name="s")


def _deg_pass(didx, np_pad):
    ep = didx.shape[0]
    nblk = ep // (K * NW)
    ones_host = jnp.ones((K, 128), jnp.float32)
    zeros_host = jnp.zeros((ZCH, 128), jnp.float32)

    @functools.partial(
        pl.kernel,
        out_type=jax.ShapeDtypeStruct((NC, np_pad, 128), jnp.float32),
        mesh=_mesh(),
        scratch_types=[
            pltpu.VMEM((K,), jnp.int32),
            pltpu.VMEM((K, 128), jnp.float32),
            pltpu.VMEM((ZCH, 128), jnp.float32),
            pltpu.VMEM_SHARED((np_pad, 128), jnp.float32),
        ],
    )
    def kern(didx_hbm, ones_hbm, zeros_hbm, out_hbm, dbuf, ones_v, zbuf, acc):
        cid = lax.axis_index("c")
        sid = lax.axis_index("s")
        wid = sid * NC + cid
        pltpu.sync_copy(ones_hbm, ones_v)
        pltpu.sync_copy(zeros_hbm, zbuf)

        @pl.loop(0, np_pad // (ZCH * NS))
        def _(i):
            pltpu.sync_copy(zbuf, acc.at[pl.ds((i * NS + sid) * ZCH, ZCH)])

        plsc.subcore_barrier()
        base = wid * nblk * K

        @pl.loop(0, nblk)
        def _(b):
            pltpu.sync_copy(didx_hbm.at[pl.ds(base + b * K, K)], dbuf)
            pltpu.sync_copy(ones_v, acc.at[dbuf], add=True)

        plsc.subcore_barrier()
        rps = np_pad // NS
        pltpu.sync_copy(acc.at[pl.ds(sid * rps, rps)],
                        out_hbm.at[cid, pl.ds(sid * rps, rps)])

    return kern(didx, ones_host, zeros_host)


def _edge_pass(table, gidx, didx, np_pad):
    ep = gidx.shape[0]
    h = table.shape[1]
    nblk = ep // (K * NW)
    zeros_host = jnp.zeros((ZCH, h), jnp.float32)

    @functools.partial(
        pl.kernel,
        out_type=jax.ShapeDtypeStruct((NC, np_pad, h), jnp.float32),
        mesh=_mesh(),
        scratch_types=[
            pltpu.VMEM((K,), jnp.int32),
            pltpu.VMEM((K,), jnp.int32),
            pltpu.VMEM((K, h), jnp.float32),
            pltpu.VMEM((ZCH, h), jnp.float32),
            pltpu.VMEM_SHARED((np_pad, h), jnp.float32),
            pltpu.SemaphoreType.DMA,
        ],
    )
    def kern(table_hbm, gidx_hbm, didx_hbm, zeros_hbm, out_hbm,
             gbuf, dbuf, rows, zbuf, acc, sem):
        cid = lax.axis_index("c")
        sid = lax.axis_index("s")
        wid = sid * NC + cid
        pltpu.sync_copy(zeros_hbm, zbuf)

        @pl.loop(0, np_pad // (ZCH * NS))
        def _(i):
            pltpu.sync_copy(zbuf, acc.at[pl.ds((i * NS + sid) * ZCH, ZCH)])

        plsc.subcore_barrier()
        base = wid * nblk * K

        @pl.loop(0, nblk)
        def _(b):
            off = base + b * K
            pltpu.sync_copy(gidx_hbm.at[pl.ds(off, K)], gbuf)
            pltpu.sync_copy(didx_hbm.at[pl.ds(off, K)], dbuf)
            pltpu.async_copy(table_hbm.at[gbuf], rows, sem).wait()
            pltpu.sync_copy(rows, acc.at[dbuf], add=True)

        plsc.subcore_barrier()
        rps = np_pad // NS
        pltpu.sync_copy(acc.at[pl.ds(sid * rps, rps)],
                        out_hbm.at[cid, pl.ds(sid * rps, rps)])

    return kern(table, gidx, didx, zeros_host)


def _aspect_gather(hmat, aidx):
    a = aidx.shape[0]
    h = hmat.shape[1]
    apt = a // NW

    @functools.partial(
        pl.kernel,
        out_type=jax.ShapeDtypeStruct((a, h), jnp.float32),
        mesh=_mesh(),
        scratch_types=[
            pltpu.VMEM((apt,), jnp.int32),
            pltpu.VMEM((apt, h), jnp.float32),
            pltpu.SemaphoreType.DMA,
        ],
    )
    def kern(h_hbm, aidx_hbm, out_hbm, ibuf, rows, sem):
        cid = lax.axis_index("c")
        sid = lax.axis_index("s")
        wid = sid * NC + cid
        pltpu.sync_copy(aidx_hbm.at[pl.ds(wid * apt, apt)], ibuf)
        pltpu.async_copy(h_hbm.at[ibuf], rows, sem).wait()
        pltpu.sync_copy(rows, out_hbm.at[pl.ds(wid * apt, apt)])

    return kern(hmat, aidx)


def _dinv_kernel(degs):
    np_pad = degs.shape[1]

    def body(d_ref, o_ref):
        deg = d_ref[0, :, 0:1] + d_ref[1, :, 0:1]
        o_ref[...] = lax.rsqrt(jnp.maximum(deg, 1.0))

    return pl.pallas_call(
        body,
        out_shape=jax.ShapeDtypeStruct((np_pad, 1), jnp.float32),
    )(degs)


def _typed_transform(x, w, dinv, bn=2048):
    np_pad, d = x.shape
    t, _, h = w.shape

    def body(x_ref, w_ref, dv_ref, o_ref):
        o_ref[0] = dv_ref[...] * jnp.dot(
            x_ref[...], w_ref[0], preferred_element_type=jnp.float32)

    return pl.pallas_call(
        body,
        grid=(t, np_pad // bn),
        in_specs=[
            pl.BlockSpec((bn, d), lambda ti, i: (i, 0)),
            pl.BlockSpec((1, d, h), lambda ti, i: (ti, 0, 0)),
            pl.BlockSpec((bn, 1), lambda ti, i: (i, 0)),
        ],
        out_specs=pl.BlockSpec((1, bn, h), lambda ti, i: (ti, i, 0)),
        out_shape=jax.ShapeDtypeStruct((t, np_pad, h), jnp.float32),
    )(x, w, dinv)


def _root_matmul(x, w_root, b, bn=2048):
    np_pad, d = x.shape
    h = w_root.shape[1]

    def body(x_ref, w_ref, b_ref, o_ref):
        o_ref[...] = jnp.dot(
            x_ref[...], w_ref[...], preferred_element_type=jnp.float32
        ) + b_ref[...]

    return pl.pallas_call(
        body,
        grid=(np_pad // bn,),
        in_specs=[
            pl.BlockSpec((bn, d), lambda i: (i, 0)),
            pl.BlockSpec((d, h), lambda i: (0, 0)),
            pl.BlockSpec((1, h), lambda i: (0, 0)),
        ],
        out_specs=pl.BlockSpec((bn, h), lambda i: (i, 0)),
        out_shape=jax.ShapeDtypeStruct((np_pad, h), jnp.float32),
    )(x, w_root, b)


def _combine(acc, dinv, root, relu, bn=2048):
    np_pad, h = root.shape

    def body(a_ref, dv_ref, r_ref, o_ref):
        s = (a_ref[0] + a_ref[1]) * dv_ref[...] + r_ref[...]
        if relu:
            s = jnp.maximum(s, 0.0)
        o_ref[...] = s

    return pl.pallas_call(
        body,
        grid=(np_pad // bn,),
        in_specs=[
            pl.BlockSpec((2, bn, h), lambda i: (0, i, 0)),
            pl.BlockSpec((bn, 1), lambda i: (i, 0)),
            pl.BlockSpec((bn, h), lambda i: (i, 0)),
        ],
        out_specs=pl.BlockSpec((bn, h), lambda i: (i, 0)),
        out_shape=jax.ShapeDtypeStruct((np_pad, h), jnp.float32),
    )(acc, dinv, root)


def _classifier(asp, wc, bc):
    a, h = asp.shape
    c = wc.shape[1]

    def body(x_ref, w_ref, b_ref, o_ref):
        o_ref[...] = jnp.dot(
            x_ref[...], w_ref[...], preferred_element_type=jnp.float32
        ) + b_ref[...]

    return pl.pallas_call(
        body,
        out_shape=jax.ShapeDtypeStruct((a, c), jnp.float32),
    )(asp, wc, bc.reshape(1, c))


def kernel(features, edge_index, aspect_indices, edge_types,
           W1, W1_root, b1, W2, W2_root, b2, Wc, bc):
    n, d = features.shape
    e = edge_index.shape[1]
    t = W1.shape[0]
    h = W1.shape[2]

    np_pad = -(-(n + 1) // (ZCH * NS)) * (ZCH * NS)
    e_pad = -(-e // (K * NW)) * (K * NW)

    src = edge_index[0].astype(jnp.int32)
    dst = edge_index[1].astype(jnp.int32)
    et = edge_types.astype(jnp.int32)
    pad = e_pad - e
    gidx = jnp.concatenate([et * np_pad + src,
                            jnp.zeros((pad,), jnp.int32)])
    didx = jnp.concatenate([dst, jnp.full((pad,), n, jnp.int32)])
    aidx = aspect_indices.astype(jnp.int32)

    x = jnp.pad(features, ((0, np_pad - n), (0, 0)))

    degs = _deg_pass(didx, np_pad)
    dinv = _dinv_kernel(degs)

    xts1 = _typed_transform(x, W1, dinv).reshape(t * np_pad, h)
    acc1 = _edge_pass(xts1, gidx, didx, np_pad)
    root1 = _root_matmul(x, W1_root, b1.reshape(1, h))
    h1 = _combine(acc1, dinv, root1, relu=True)

    xts2 = _typed_transform(h1, W2, dinv).reshape(t * np_pad, h)
    acc2 = _edge_pass(xts2, gidx, didx, np_pad)
    root2 = _root_matmul(h1, W2_root, b2.reshape(1, h))
    h2 = _combine(acc2, dinv, root2, relu=False)

    asp = _aspect_gather(h2, aidx)
    return _classifier(asp, Wc, bc)
